# baseline (speedup 1.0000x reference)
import jax
import jax.numpy as jnp
from jax import lax
from jax.experimental import pallas as pl
from jax.experimental.pallas import tpu as pltpu
from jax.experimental.pallas import tpu_sc as plsc

_NTILE = 16
_NCORE = 2
_LANES = 16
_EDGE_BATCH = 128


def _row_split(n_nodes):
  rows_base = (n_nodes // (_NTILE * 8)) * 8
  extra = n_nodes - _NTILE * rows_base
  assert extra % 8 == 0
  return rows_base, extra


def _make_seg_sum(n_nodes: int, n_edges: int, half: int):
  B = _EDGE_BATCH
  per_tile = n_edges // _NTILE
  assert per_tile * _NTILE == n_edges
  nbatch = per_tile // B
  tail = per_tile - nbatch * B
  assert nbatch >= 2 and tail % _LANES == 0
  rows_base, extra = _row_split(n_nodes)
  ZR = max(z for z in range(8, 49, 8) if rows_base % z == 0)
  nzcopy = rows_base // ZR
  assert extra <= ZR

  mesh = plsc.VectorSubcoreMesh(core_axis_name="c", subcore_axis_name="s")
  out_type = jax.ShapeDtypeStruct((_NCORE, n_nodes, half), jnp.float32)
  scratch = [
      pltpu.VMEM((B,), jnp.int32),
      pltpu.VMEM((B,), jnp.int32),
      pltpu.VMEM((B,), jnp.int32),
      pltpu.VMEM((B,), jnp.int32),
      pltpu.VMEM((B, half), jnp.float32),
      pltpu.VMEM((B, half), jnp.float32),
      pltpu.VMEM((ZR, half), jnp.float32),
      pltpu.VMEM_SHARED((n_nodes, half), jnp.float32),
      pltpu.SemaphoreType.DMA,
      pltpu.SemaphoreType.DMA,
      pltpu.SemaphoreType.DMA,
      pltpu.SemaphoreType.DMA,
      pltpu.SemaphoreType.DMA,
  ]
  if tail:
    scratch += [
        pltpu.VMEM((tail,), jnp.int32),
        pltpu.VMEM((tail,), jnp.int32),
        pltpu.VMEM((tail, half), jnp.float32),
    ]

  def body(src_hbm, dst_hbm, x2_hbm, sums_hbm, srcv0, srcv1, dstv0, dstv1,
           rows0, rows1, zero_v, acc_sh, sg0, sg1, ss0, ss1, si, *tbufs):
    srcv = (srcv0, srcv1)
    dstv = (dstv0, dstv1)
    rows = (rows0, rows1)
    sg = (sg0, sg1)
    ss = (ss0, ss1)
    c = lax.axis_index("c")
    s = lax.axis_index("s")

    def zrow(r, _):
      def zlane(k, _):
        zero_v[r, pl.ds(k * _LANES, _LANES)] = jnp.zeros(
            (_LANES,), jnp.float32)
        return 0
      return lax.fori_loop(0, half // _LANES, zlane, 0)
    lax.fori_loop(0, ZR, zrow, 0)

    r0 = s * rows_base
    for z in range(nzcopy):
      pltpu.sync_copy(zero_v, acc_sh.at[pl.ds(r0 + z * ZR, ZR)])
    if extra:
      @pl.when(s == _NTILE - 1)
      def _():
        pltpu.sync_copy(zero_v.at[pl.ds(0, extra)],
                        acc_sh.at[pl.ds(n_nodes - extra, extra)])

    plsc.subcore_barrier()

    ebase = s * per_tile

    def load_tx(jj, b):
      base = ebase + jj * B
      pltpu.async_copy(src_hbm.at[pl.ds(base, B)], srcv[b], si)
      pltpu.async_copy(dst_hbm.at[pl.ds(base, B)], dstv[b], si)
      pltpu.make_async_copy(src_hbm.at[pl.ds(base, B)], srcv[b], si).wait()
      pltpu.make_async_copy(dst_hbm.at[pl.ds(base, B)], dstv[b], si).wait()

      def tx(k, _):
        v = srcv[b][pl.ds(k * _LANES, _LANES)]
        srcv[b][pl.ds(k * _LANES, _LANES)] = v * 2 + c
        return 0
      lax.fori_loop(0, B // _LANES, tx, 0)

    def issue_gather(b):
      pltpu.async_copy(x2_hbm.at[srcv[b]], rows[b], sg[b])

    def wait_gather(b):
      pltpu.make_async_copy(x2_hbm.at[srcv[b]], rows[b], sg[b]).wait()

    def issue_scatter(b):
      pltpu.async_copy(rows[b], acc_sh.at[dstv[b]], ss[b], add=True)

    def wait_scatter(b):
      pltpu.make_async_copy(rows[b], acc_sh.at[dstv[b]], ss[b]).wait()

    load_tx(0, 0)
    issue_gather(0)
    load_tx(1, 1)
    wait_gather(0)
    issue_gather(1)
    issue_scatter(0)

    def step(g, _):
      for off, b in ((1, 1), (2, 0)):
        jj = 2 * g + off
        o = 1 - b
        wait_gather(b)
        wait_scatter(o)
        issue_scatter(b)

        @pl.when(jj < nbatch - 1)
        def _():
          load_tx(jj + 1, o)
          issue_gather(o)
      return 0
    lax.fori_loop(0, (nbatch - 1) // 2, step, 0)

    if (nbatch - 1) % 2:
      b = (nbatch - 1) % 2
      wait_gather(b)
      wait_scatter(1 - b)
      issue_scatter(b)

    wait_scatter((nbatch - 1) % 2)

    if tail:
      tsrc, tdst, trows = tbufs
      base = ebase + nbatch * B
      pltpu.sync_copy(src_hbm.at[pl.ds(base, tail)], tsrc)
      pltpu.sync_copy(dst_hbm.at[pl.ds(base, tail)], tdst)

      def ttx(k, _):
        v = tsrc[pl.ds(k * _LANES, _LANES)]
        tsrc[pl.ds(k * _LANES, _LANES)] = v * 2 + c
        return 0
      lax.fori_loop(0, tail // _LANES, ttx, 0)
      pltpu.async_copy(x2_hbm.at[tsrc], trows, si).wait()
      pltpu.sync_copy(trows, acc_sh.at[tdst], add=True)

    plsc.subcore_barrier()

    pltpu.sync_copy(acc_sh.at[pl.ds(r0, rows_base)],
                    sums_hbm.at[c, pl.ds(r0, rows_base)])
    if extra:
      @pl.when(s == _NTILE - 1)
      def _():
        pltpu.sync_copy(acc_sh.at[pl.ds(n_nodes - extra, extra)],
                        sums_hbm.at[c, pl.ds(n_nodes - extra, extra)])

  return pl.kernel(body, out_type=out_type, mesh=mesh,
                   scratch_types=tuple(scratch))


def _make_cnt(n_nodes: int, n_edges: int, half: int):
  per_tile = n_edges // (_NTILE * _NCORE)
  assert per_tile * _NTILE * _NCORE == n_edges
  CB = 128
  nbatch = per_tile // CB
  tail = per_tile - nbatch * CB
  assert nbatch >= 2 and tail % 8 == 0
  rows_base, extra = _row_split(n_nodes)
  ZR = max(z for z in range(8, 49, 8) if rows_base % z == 0)
  nzcopy = rows_base // ZR
  assert extra <= ZR

  mesh = plsc.VectorSubcoreMesh(core_axis_name="c", subcore_axis_name="s")
  out_type = jax.ShapeDtypeStruct((_NCORE, n_nodes, half), jnp.float32)
  scratch = [
      pltpu.VMEM((CB,), jnp.int32),
      pltpu.VMEM((CB,), jnp.int32),
      pltpu.VMEM((CB, half), jnp.float32),
      pltpu.VMEM((ZR, half), jnp.float32),
      pltpu.VMEM_SHARED((n_nodes, half), jnp.float32),
      pltpu.SemaphoreType.DMA,
      pltpu.SemaphoreType.DMA,
  ]
  if tail:
    scratch.append(pltpu.VMEM((tail,), jnp.int32))

  def body(dst_hbm, cnt_hbm, idx0, idx1, ones_v, zero_v, cnt_sh,
           ss0, ss1, *tbufs):
    c = lax.axis_index("c")
    s = lax.axis_index("s")
    idx = (idx0, idx1)
    ss = (ss0, ss1)

    def zrow(r, _):
      def zlane(k, _):
        zero_v[r, pl.ds(k * _LANES, _LANES)] = jnp.zeros(
            (_LANES,), jnp.float32)
        return 0
      return lax.fori_loop(0, half // _LANES, zlane, 0)
    lax.fori_loop(0, ZR, zrow, 0)

    def orow(r, _):
      def olane(k, _):
        ones_v[r, pl.ds(k * _LANES, _LANES)] = jnp.ones(
            (_LANES,), jnp.float32)
        return 0
      return lax.fori_loop(0, half // _LANES, olane, 0)
    lax.fori_loop(0, CB, orow, 0)

    r0 = s * rows_base
    for z in range(nzcopy):
      pltpu.sync_copy(zero_v, cnt_sh.at[pl.ds(r0 + z * ZR, ZR)])
    if extra:
      @pl.when(s == _NTILE - 1)
      def _():
        pltpu.sync_copy(zero_v.at[pl.ds(0, extra)],
                        cnt_sh.at[pl.ds(n_nodes - extra, extra)])

    plsc.subcore_barrier()

    ebase = (c * _NTILE + s) * per_tile

    def load_idx(jj, b):
      pltpu.sync_copy(dst_hbm.at[pl.ds(ebase + jj * CB, CB)], idx[b])

    def issue_scatter(b):
      pltpu.async_copy(ones_v, cnt_sh.at[idx[b]], ss[b], add=True)

    def wait_scatter(b):
      pltpu.make_async_copy(ones_v, cnt_sh.at[idx[b]], ss[b]).wait()

    load_idx(0, 0)
    issue_scatter(0)
    load_idx(1, 1)

    def step(g, _):
      for off, b in ((1, 1), (2, 0)):
        jj = 2 * g + off
        o = 1 - b
        wait_scatter(o)
        issue_scatter(b)

        @pl.when(jj < nbatch - 1)
        def _():
          load_idx(jj + 1, o)
      return 0
    lax.fori_loop(0, (nbatch - 1) // 2, step, 0)

    if (nbatch - 1) % 2:
      b = (nbatch - 1) % 2
      wait_scatter(1 - b)
      issue_scatter(b)

    wait_scatter((nbatch - 1) % 2)

    if tail:
      (tidx,) = tbufs
      pltpu.sync_copy(dst_hbm.at[pl.ds(ebase + nbatch * CB, tail)], tidx)
      pltpu.sync_copy(ones_v.at[pl.ds(0, tail)], cnt_sh.at[tidx], add=True)

    plsc.subcore_barrier()

    pltpu.sync_copy(cnt_sh.at[pl.ds(r0, rows_base)],
                    cnt_hbm.at[c, pl.ds(r0, rows_base)])
    if extra:
      @pl.when(s == _NTILE - 1)
      def _():
        pltpu.sync_copy(cnt_sh.at[pl.ds(n_nodes - extra, extra)],
                        cnt_hbm.at[c, pl.ds(n_nodes - extra, extra)])

  return pl.kernel(body, out_type=out_type, mesh=mesh,
                   scratch_types=tuple(scratch))


def _mm_body(sums_ref, cnt_ref, x_ref, wl_ref, b_ref, wr_ref, o_ref):
  agg = jnp.concatenate([sums_ref[0], sums_ref[1]], axis=-1)
  cnt = cnt_ref[0][:, 0:1] + cnt_ref[1][:, 0:1]
  agg = agg / jnp.maximum(cnt, 1.0)
  h = lax.dot_general(agg, wl_ref[...], (((1,), (1,)), ((), ())),
                      preferred_element_type=jnp.float32)
  h = h + lax.dot_general(x_ref[...], wr_ref[...], (((1,), (1,)), ((), ())),
                          preferred_element_type=jnp.float32)
  h = h + b_ref[...]
  o_ref[...] = jnp.maximum(h, 0.0)


def _sage_mm(sums, cnt, x, wl, b2d, wr, block_rows=1000):
  n, d = x.shape
  h = wl.shape[0]
  half = sums.shape[2]
  grid = (n // block_rows,)
  return pl.pallas_call(
      _mm_body,
      grid=grid,
      in_specs=[
          pl.BlockSpec((2, block_rows, half), lambda i: (0, i, 0)),
          pl.BlockSpec((2, block_rows, half), lambda i: (0, i, 0)),
          pl.BlockSpec((block_rows, d), lambda i: (i, 0)),
          pl.BlockSpec((h, d), lambda i: (0, 0)),
          pl.BlockSpec((1, h), lambda i: (0, 0)),
          pl.BlockSpec((h, d), lambda i: (0, 0)),
      ],
      out_specs=pl.BlockSpec((block_rows, h), lambda i: (i, 0)),
      out_shape=jax.ShapeDtypeStruct((n, h), jnp.float32),
  )(sums, cnt, x, wl, b2d, wr)


def kernel(x, edge_index, W1l, b1, W1r, W2l, b2, W2r):
  n, d = x.shape
  e = edge_index.shape[1]
  half = d // 2

  seg_sum = _make_seg_sum(n, e, half)
  cnt_fn = _make_cnt(n, e, half)

  src = edge_index[0]
  dst = edge_index[1]
  x2 = x.reshape(n * 2, half)

  cnt = cnt_fn(dst)
  sums1 = seg_sum(src, dst, x2)
  h = _sage_mm(sums1, cnt, x, W1l, b1.reshape(1, -1), W1r)

  h2 = h.reshape(n * 2, half)
  sums2 = seg_sum(src, dst, h2)
  out = _sage_mm(sums2, cnt, h, W2l, b2.reshape(1, -1), W2r)
  return out


# speedup vs baseline: 1.0147x; 1.0147x over previous
import jax
import jax.numpy as jnp
from jax import lax
from jax.experimental import pallas as pl
from jax.experimental.pallas import tpu as pltpu
from jax.experimental.pallas import tpu_sc as plsc

_NTILE = 16
_NCORE = 2
_LANES = 16
_EDGE_BATCH = 128


def _row_split(n_nodes):
  rows_base = (n_nodes // (_NTILE * 8)) * 8
  extra = n_nodes - _NTILE * rows_base
  assert extra % 8 == 0
  return rows_base, extra


def _make_seg_sum(n_nodes: int, n_edges: int, half: int):
  B = _EDGE_BATCH
  per_tile = n_edges // _NTILE
  assert per_tile * _NTILE == n_edges
  nbatch = per_tile // B
  tail = per_tile - nbatch * B
  assert nbatch >= 2 and tail % _LANES == 0
  rows_base, extra = _row_split(n_nodes)
  ZR = max(z for z in range(8, 49, 8) if rows_base % z == 0)
  nzcopy = rows_base // ZR
  assert extra <= ZR

  mesh = plsc.VectorSubcoreMesh(core_axis_name="c", subcore_axis_name="s")
  out_type = jax.ShapeDtypeStruct((_NCORE, n_nodes, half), jnp.float32)
  scratch = [
      pltpu.VMEM((B,), jnp.int32),
      pltpu.VMEM((B,), jnp.int32),
      pltpu.VMEM((B,), jnp.int32),
      pltpu.VMEM((B,), jnp.int32),
      pltpu.VMEM((B, half), jnp.float32),
      pltpu.VMEM((B, half), jnp.float32),
      pltpu.VMEM((ZR, half), jnp.float32),
      pltpu.VMEM_SHARED((n_nodes, half), jnp.float32),
      pltpu.SemaphoreType.DMA,
      pltpu.SemaphoreType.DMA,
      pltpu.SemaphoreType.DMA,
      pltpu.SemaphoreType.DMA,
      pltpu.SemaphoreType.DMA,
      pltpu.SemaphoreType.DMA,
  ]
  if tail:
    scratch += [
        pltpu.VMEM((tail,), jnp.int32),
        pltpu.VMEM((tail,), jnp.int32),
        pltpu.VMEM((tail, half), jnp.float32),
    ]

  def body(src_hbm, dst_hbm, x2_hbm, sums_hbm, srcv0, srcv1, dstv0, dstv1,
           rows0, rows1, zero_v, acc_sh, sg0, sg1, ss0, ss1, si, sz, *tbufs):
    srcv = (srcv0, srcv1)
    dstv = (dstv0, dstv1)
    rows = (rows0, rows1)
    sg = (sg0, sg1)
    ss = (ss0, ss1)
    c = lax.axis_index("c")
    s = lax.axis_index("s")

    def zrow(r, _):
      def zlane(k, _):
        zero_v[r, pl.ds(k * _LANES, _LANES)] = jnp.zeros(
            (_LANES,), jnp.float32)
        return 0
      return lax.fori_loop(0, half // _LANES, zlane, 0)
    lax.fori_loop(0, ZR, zrow, 0)

    r0 = s * rows_base
    for z in range(nzcopy):
      pltpu.async_copy(zero_v, acc_sh.at[pl.ds(r0 + z * ZR, ZR)], sz)
    if extra:
      @pl.when(s == _NTILE - 1)
      def _():
        pltpu.async_copy(zero_v.at[pl.ds(0, extra)],
                         acc_sh.at[pl.ds(n_nodes - extra, extra)], sz)

    ebase = s * per_tile

    def load_tx(jj, b):
      base = ebase + jj * B
      pltpu.async_copy(src_hbm.at[pl.ds(base, B)], srcv[b], si)
      pltpu.async_copy(dst_hbm.at[pl.ds(base, B)], dstv[b], si)
      pltpu.make_async_copy(src_hbm.at[pl.ds(base, B)], srcv[b], si).wait()
      pltpu.make_async_copy(dst_hbm.at[pl.ds(base, B)], dstv[b], si).wait()

      def tx(k, _):
        v = srcv[b][pl.ds(k * _LANES, _LANES)]
        srcv[b][pl.ds(k * _LANES, _LANES)] = v * 2 + c
        return 0
      lax.fori_loop(0, B // _LANES, tx, 0)

    def issue_gather(b):
      pltpu.async_copy(x2_hbm.at[srcv[b]], rows[b], sg[b])

    def wait_gather(b):
      pltpu.make_async_copy(x2_hbm.at[srcv[b]], rows[b], sg[b]).wait()

    def issue_scatter(b):
      pltpu.async_copy(rows[b], acc_sh.at[dstv[b]], ss[b], add=True)

    def wait_scatter(b):
      pltpu.make_async_copy(rows[b], acc_sh.at[dstv[b]], ss[b]).wait()

    load_tx(0, 0)
    issue_gather(0)
    load_tx(1, 1)
    for z in range(nzcopy):
      pltpu.make_async_copy(zero_v, acc_sh.at[pl.ds(r0 + z * ZR, ZR)],
                            sz).wait()
    if extra:
      @pl.when(s == _NTILE - 1)
      def _():
        pltpu.make_async_copy(zero_v.at[pl.ds(0, extra)],
                              acc_sh.at[pl.ds(n_nodes - extra, extra)],
                              sz).wait()
    plsc.subcore_barrier()
    wait_gather(0)
    issue_gather(1)
    issue_scatter(0)

    def step(g, _):
      for off, b in ((1, 1), (2, 0)):
        jj = 2 * g + off
        o = 1 - b
        wait_gather(b)
        wait_scatter(o)
        issue_scatter(b)

        @pl.when(jj < nbatch - 1)
        def _():
          load_tx(jj + 1, o)
          issue_gather(o)
      return 0
    lax.fori_loop(0, (nbatch - 1) // 2, step, 0)

    if (nbatch - 1) % 2:
      b = (nbatch - 1) % 2
      wait_gather(b)
      wait_scatter(1 - b)
      issue_scatter(b)

    wait_scatter((nbatch - 1) % 2)

    if tail:
      tsrc, tdst, trows = tbufs
      base = ebase + nbatch * B
      pltpu.sync_copy(src_hbm.at[pl.ds(base, tail)], tsrc)
      pltpu.sync_copy(dst_hbm.at[pl.ds(base, tail)], tdst)

      def ttx(k, _):
        v = tsrc[pl.ds(k * _LANES, _LANES)]
        tsrc[pl.ds(k * _LANES, _LANES)] = v * 2 + c
        return 0
      lax.fori_loop(0, tail // _LANES, ttx, 0)
      pltpu.async_copy(x2_hbm.at[tsrc], trows, si).wait()
      pltpu.sync_copy(trows, acc_sh.at[tdst], add=True)

    plsc.subcore_barrier()

    pltpu.sync_copy(acc_sh.at[pl.ds(r0, rows_base)],
                    sums_hbm.at[c, pl.ds(r0, rows_base)])
    if extra:
      @pl.when(s == _NTILE - 1)
      def _():
        pltpu.sync_copy(acc_sh.at[pl.ds(n_nodes - extra, extra)],
                        sums_hbm.at[c, pl.ds(n_nodes - extra, extra)])

  return pl.kernel(body, out_type=out_type, mesh=mesh,
                   scratch_types=tuple(scratch))


def _make_cnt(n_nodes: int, n_edges: int, half: int):
  per_tile = n_edges // (_NTILE * _NCORE)
  assert per_tile * _NTILE * _NCORE == n_edges
  CB = 128
  nbatch = per_tile // CB
  tail = per_tile - nbatch * CB
  assert nbatch >= 2 and tail % 8 == 0
  rows_base, extra = _row_split(n_nodes)
  ZR = max(z for z in range(8, 49, 8) if rows_base % z == 0)
  nzcopy = rows_base // ZR
  assert extra <= ZR

  mesh = plsc.VectorSubcoreMesh(core_axis_name="c", subcore_axis_name="s")
  out_type = jax.ShapeDtypeStruct((_NCORE, n_nodes, half), jnp.float32)
  scratch = [
      pltpu.VMEM((CB,), jnp.int32),
      pltpu.VMEM((CB,), jnp.int32),
      pltpu.VMEM((CB, half), jnp.float32),
      pltpu.VMEM((ZR, half), jnp.float32),
      pltpu.VMEM_SHARED((n_nodes, half), jnp.float32),
      pltpu.SemaphoreType.DMA,
      pltpu.SemaphoreType.DMA,
      pltpu.SemaphoreType.DMA,
  ]
  if tail:
    scratch.append(pltpu.VMEM((tail,), jnp.int32))

  def body(dst_hbm, cnt_hbm, idx0, idx1, ones_v, zero_v, cnt_sh,
           ss0, ss1, sz, *tbufs):
    c = lax.axis_index("c")
    s = lax.axis_index("s")
    idx = (idx0, idx1)
    ss = (ss0, ss1)

    def zrow(r, _):
      def zlane(k, _):
        zero_v[r, pl.ds(k * _LANES, _LANES)] = jnp.zeros(
            (_LANES,), jnp.float32)
        return 0
      return lax.fori_loop(0, half // _LANES, zlane, 0)
    lax.fori_loop(0, ZR, zrow, 0)

    def orow(r, _):
      def olane(k, _):
        ones_v[r, pl.ds(k * _LANES, _LANES)] = jnp.ones(
            (_LANES,), jnp.float32)
        return 0
      return lax.fori_loop(0, half // _LANES, olane, 0)
    lax.fori_loop(0, CB, orow, 0)

    r0 = s * rows_base
    for z in range(nzcopy):
      pltpu.async_copy(zero_v, cnt_sh.at[pl.ds(r0 + z * ZR, ZR)], sz)
    if extra:
      @pl.when(s == _NTILE - 1)
      def _():
        pltpu.async_copy(zero_v.at[pl.ds(0, extra)],
                         cnt_sh.at[pl.ds(n_nodes - extra, extra)], sz)

    ebase = (c * _NTILE + s) * per_tile

    def load_idx(jj, b):
      pltpu.sync_copy(dst_hbm.at[pl.ds(ebase + jj * CB, CB)], idx[b])

    def issue_scatter(b):
      pltpu.async_copy(ones_v, cnt_sh.at[idx[b]], ss[b], add=True)

    def wait_scatter(b):
      pltpu.make_async_copy(ones_v, cnt_sh.at[idx[b]], ss[b]).wait()

    load_idx(0, 0)
    for z in range(nzcopy):
      pltpu.make_async_copy(zero_v, cnt_sh.at[pl.ds(r0 + z * ZR, ZR)],
                            sz).wait()
    if extra:
      @pl.when(s == _NTILE - 1)
      def _():
        pltpu.make_async_copy(zero_v.at[pl.ds(0, extra)],
                              cnt_sh.at[pl.ds(n_nodes - extra, extra)],
                              sz).wait()
    plsc.subcore_barrier()
    issue_scatter(0)
    load_idx(1, 1)

    def step(g, _):
      for off, b in ((1, 1), (2, 0)):
        jj = 2 * g + off
        o = 1 - b
        wait_scatter(o)
        issue_scatter(b)

        @pl.when(jj < nbatch - 1)
        def _():
          load_idx(jj + 1, o)
      return 0
    lax.fori_loop(0, (nbatch - 1) // 2, step, 0)

    if (nbatch - 1) % 2:
      b = (nbatch - 1) % 2
      wait_scatter(1 - b)
      issue_scatter(b)

    wait_scatter((nbatch - 1) % 2)

    if tail:
      (tidx,) = tbufs
      pltpu.sync_copy(dst_hbm.at[pl.ds(ebase + nbatch * CB, tail)], tidx)
      pltpu.sync_copy(ones_v.at[pl.ds(0, tail)], cnt_sh.at[tidx], add=True)

    plsc.subcore_barrier()

    pltpu.sync_copy(cnt_sh.at[pl.ds(r0, rows_base)],
                    cnt_hbm.at[c, pl.ds(r0, rows_base)])
    if extra:
      @pl.when(s == _NTILE - 1)
      def _():
        pltpu.sync_copy(cnt_sh.at[pl.ds(n_nodes - extra, extra)],
                        cnt_hbm.at[c, pl.ds(n_nodes - extra, extra)])

  return pl.kernel(body, out_type=out_type, mesh=mesh,
                   scratch_types=tuple(scratch))


def _mm_body(sums_ref, cnt_ref, x_ref, wl_ref, b_ref, wr_ref, o_ref):
  agg = jnp.concatenate([sums_ref[0], sums_ref[1]], axis=-1)
  cnt = cnt_ref[0][:, 0:1] + cnt_ref[1][:, 0:1]
  agg = agg / jnp.maximum(cnt, 1.0)
  h = lax.dot_general(agg, wl_ref[...], (((1,), (1,)), ((), ())),
                      preferred_element_type=jnp.float32)
  h = h + lax.dot_general(x_ref[...], wr_ref[...], (((1,), (1,)), ((), ())),
                          preferred_element_type=jnp.float32)
  h = h + b_ref[...]
  o_ref[...] = jnp.maximum(h, 0.0)


def _sage_mm(sums, cnt, x, wl, b2d, wr, block_rows=1000):
  n, d = x.shape
  h = wl.shape[0]
  half = sums.shape[2]
  grid = (n // block_rows,)
  return pl.pallas_call(
      _mm_body,
      grid=grid,
      in_specs=[
          pl.BlockSpec((2, block_rows, half), lambda i: (0, i, 0)),
          pl.BlockSpec((2, block_rows, half), lambda i: (0, i, 0)),
          pl.BlockSpec((block_rows, d), lambda i: (i, 0)),
          pl.BlockSpec((h, d), lambda i: (0, 0)),
          pl.BlockSpec((1, h), lambda i: (0, 0)),
          pl.BlockSpec((h, d), lambda i: (0, 0)),
      ],
      out_specs=pl.BlockSpec((block_rows, h), lambda i: (i, 0)),
      out_shape=jax.ShapeDtypeStruct((n, h), jnp.float32),
  )(sums, cnt, x, wl, b2d, wr)


def kernel(x, edge_index, W1l, b1, W1r, W2l, b2, W2r):
  n, d = x.shape
  e = edge_index.shape[1]
  half = d // 2

  seg_sum = _make_seg_sum(n, e, half)
  cnt_fn = _make_cnt(n, e, half)

  src = edge_index[0]
  dst = edge_index[1]
  x2 = x.reshape(n * 2, half)

  cnt = cnt_fn(dst)
  sums1 = seg_sum(src, dst, x2)
  h = _sage_mm(sums1, cnt, x, W1l, b1.reshape(1, -1), W1r)

  h2 = h.reshape(n * 2, half)
  sums2 = seg_sum(src, dst, h2)
  out = _sage_mm(sums2, cnt, h, W2l, b2.reshape(1, -1), W2r)
  return out
